# baseline (speedup 1.0000x reference)
import functools

import jax
import jax.numpy as jnp
from jax import lax
from jax.experimental import pallas as pl
from jax.experimental.pallas import tpu as pltpu

E = 64
TOPK = 8
D = 768
F = 256
T = 1024
EPG = 8
GRID = E // EPG


def _moe_body(x_ref, rw_ref, wg_ref, wu_ref, wd_ref, out_ref, combine_ref,
              xb_ref, h_ref):
    step = pl.program_id(0)

    @pl.when(step == 0)
    def _routing():
        x = x_ref[...]
        xb_ref[...] = x.astype(jnp.bfloat16)
        logits = jnp.dot(x, rw_ref[...], preferred_element_type=jnp.float32)
        m = jnp.max(logits, axis=-1, keepdims=True)
        ex = jnp.exp(logits - m)
        probs = ex / jnp.sum(ex, axis=-1, keepdims=True)

        lane = lax.broadcasted_iota(jnp.int32, (T, E), 1)
        p = probs
        sel_w = jnp.zeros((T, E), jnp.float32)
        for _ in range(TOPK):
            mx = jnp.max(p, axis=-1, keepdims=True)
            eq = p >= mx
            first_idx = jnp.min(jnp.where(eq, lane, E), axis=-1, keepdims=True)
            pick = lane == first_idx
            sel_w = jnp.where(pick, probs, sel_w)
            p = jnp.where(pick, -jnp.inf, p)
        denom = jnp.sum(sel_w, axis=-1, keepdims=True)
        combine_ref[...] = sel_w / denom
        out_ref[...] = jnp.zeros((T, D), jnp.float32)

    parity = lax.rem(step, 2)

    @pl.when(step < GRID)
    def _gate_up():
        xb = xb_ref[...]
        lane = lax.broadcasted_iota(jnp.int32, (1, E), 1)
        for j in range(EPG):
            e = step * EPG + j
            g = jnp.dot(xb, wg_ref[j].astype(jnp.bfloat16),
                        preferred_element_type=jnp.float32)
            u = jnp.dot(xb, wu_ref[j].astype(jnp.bfloat16),
                        preferred_element_type=jnp.float32)
            c = jnp.sum(combine_ref[...] * (lane == e).astype(jnp.float32),
                        axis=-1, keepdims=True)
            h_ref[parity, j] = (
                (g / (1.0 + jnp.exp(-g))) * u * c).astype(jnp.bfloat16)

    @pl.when(step > 0)
    def _down():
        acc = None
        for j in range(EPG):
            y = jnp.dot(h_ref[1 - parity, j], wd_ref[j].astype(jnp.bfloat16),
                        preferred_element_type=jnp.float32)
            acc = y if acc is None else acc + y
        out_ref[...] += acc


@functools.partial(jax.jit, static_argnames=())
def kernel(hidden_states, router_w, w_gate, w_up, w_down):
    x = hidden_states.reshape(-1, D)
    last = GRID - 1
    out = pl.pallas_call(
        _moe_body,
        grid=(GRID + 1,),
        in_specs=[
            pl.BlockSpec((T, D), lambda s: (0, 0)),
            pl.BlockSpec((D, E), lambda s: (0, 0)),
            pl.BlockSpec((EPG, D, F), lambda s: (jnp.minimum(s, last), 0, 0)),
            pl.BlockSpec((EPG, D, F), lambda s: (jnp.minimum(s, last), 0, 0)),
            pl.BlockSpec((EPG, F, D),
                         lambda s: (jnp.maximum(s - 1, 0), 0, 0)),
        ],
        out_specs=pl.BlockSpec((T, D), lambda s: (0, 0)),
        out_shape=jax.ShapeDtypeStruct((T, D), jnp.float32),
        scratch_shapes=[
            pltpu.VMEM((T, E), jnp.float32),
            pltpu.VMEM((T, D), jnp.bfloat16),
            pltpu.VMEM((2, EPG, T, F), jnp.bfloat16),
        ],
    )(x, router_w, w_gate, w_up, w_down)
    return out.reshape(hidden_states.shape)


# speedup vs baseline: 1.0107x; 1.0107x over previous
import functools

import jax
import jax.numpy as jnp
from jax import lax
from jax.experimental import pallas as pl
from jax.experimental.pallas import tpu as pltpu

E = 64
TOPK = 8
D = 768
F = 256
T = 1024
EPG = 4
GRID = E // EPG


def _moe_body(x_ref, rw_ref, wg_ref, wu_ref, wd_ref, out_ref, combine_ref,
              xb_ref, h_ref):
    step = pl.program_id(0)

    @pl.when(step == 0)
    def _routing():
        x = x_ref[...]
        xb_ref[...] = x.astype(jnp.bfloat16)
        logits = jnp.dot(x, rw_ref[...], preferred_element_type=jnp.float32)
        m = jnp.max(logits, axis=-1, keepdims=True)
        ex = jnp.exp(logits - m)
        probs = ex / jnp.sum(ex, axis=-1, keepdims=True)

        lane = lax.broadcasted_iota(jnp.int32, (T, E), 1)
        p = probs
        sel_w = jnp.zeros((T, E), jnp.float32)
        for _ in range(TOPK):
            mx = jnp.max(p, axis=-1, keepdims=True)
            eq = p >= mx
            first_idx = jnp.min(jnp.where(eq, lane, E), axis=-1, keepdims=True)
            pick = lane == first_idx
            sel_w = jnp.where(pick, probs, sel_w)
            p = jnp.where(pick, -jnp.inf, p)
        denom = jnp.sum(sel_w, axis=-1, keepdims=True)
        combine_ref[...] = sel_w / denom
        out_ref[...] = jnp.zeros((T, D), jnp.float32)

    parity = lax.rem(step, 2)

    @pl.when(step < GRID)
    def _gate_up():
        xb = xb_ref[...]
        lane = lax.broadcasted_iota(jnp.int32, (1, E), 1)
        for j in range(EPG):
            e = step * EPG + j
            g = jnp.dot(xb, wg_ref[j].astype(jnp.bfloat16),
                        preferred_element_type=jnp.float32)
            u = jnp.dot(xb, wu_ref[j].astype(jnp.bfloat16),
                        preferred_element_type=jnp.float32)
            c = jnp.sum(combine_ref[...] * (lane == e).astype(jnp.float32),
                        axis=-1, keepdims=True)
            h_ref[parity, :, j * F:(j + 1) * F] = (
                (g / (1.0 + jnp.exp(-g))) * u * c).astype(jnp.bfloat16)

    @pl.when(step > 0)
    def _down():
        wd = wd_ref[...].reshape(EPG * F, D).astype(jnp.bfloat16)
        y = jnp.dot(h_ref[1 - parity], wd, preferred_element_type=jnp.float32)
        out_ref[...] += y


@functools.partial(jax.jit, static_argnames=())
def kernel(hidden_states, router_w, w_gate, w_up, w_down):
    x = hidden_states.reshape(-1, D)
    last = GRID - 1
    out = pl.pallas_call(
        _moe_body,
        grid=(GRID + 1,),
        in_specs=[
            pl.BlockSpec((T, D), lambda s: (0, 0)),
            pl.BlockSpec((D, E), lambda s: (0, 0)),
            pl.BlockSpec((EPG, D, F), lambda s: (jnp.minimum(s, last), 0, 0)),
            pl.BlockSpec((EPG, D, F), lambda s: (jnp.minimum(s, last), 0, 0)),
            pl.BlockSpec((EPG, F, D),
                         lambda s: (jnp.maximum(s - 1, 0), 0, 0)),
        ],
        out_specs=pl.BlockSpec((T, D), lambda s: (0, 0)),
        out_shape=jax.ShapeDtypeStruct((T, D), jnp.float32),
        scratch_shapes=[
            pltpu.VMEM((T, E), jnp.float32),
            pltpu.VMEM((T, D), jnp.bfloat16),
            pltpu.VMEM((2, T, EPG * F), jnp.bfloat16),
        ],
    )(x, router_w, w_gate, w_up, w_down)
    return out.reshape(hidden_states.shape)


# speedup vs baseline: 1.0145x; 1.0037x over previous
import functools

import jax
import jax.numpy as jnp
from jax import lax
from jax.experimental import pallas as pl
from jax.experimental.pallas import tpu as pltpu

E = 64
TOPK = 8
D = 768
F = 256
T = 1024
EPG = 4
GRID = E // EPG


def _moe_body(x_ref, rw_ref, wg_ref, wu_ref, wd_ref, out_ref, combine_ref,
              xb_ref, h_ref, wgu_ref):
    step = pl.program_id(0)

    @pl.when(step == 0)
    def _routing():
        x = x_ref[...]
        xb_ref[...] = x.astype(jnp.bfloat16)
        logits = jnp.dot(x, rw_ref[...], preferred_element_type=jnp.float32)
        m = jnp.max(logits, axis=-1, keepdims=True)
        ex = jnp.exp(logits - m)
        probs = ex / jnp.sum(ex, axis=-1, keepdims=True)

        lane = lax.broadcasted_iota(jnp.int32, (T, E), 1)
        p = probs
        sel_w = jnp.zeros((T, E), jnp.float32)
        for _ in range(TOPK):
            mx = jnp.max(p, axis=-1, keepdims=True)
            eq = p >= mx
            first_idx = jnp.min(jnp.where(eq, lane, E), axis=-1, keepdims=True)
            pick = lane == first_idx
            sel_w = jnp.where(pick, probs, sel_w)
            p = jnp.where(pick, -jnp.inf, p)
        denom = jnp.sum(sel_w, axis=-1, keepdims=True)
        combine_ref[...] = sel_w / denom
        out_ref[...] = jnp.zeros((T, D), jnp.float32)

    parity = lax.rem(step, 2)

    @pl.when(step < GRID)
    def _gate_up():
        xb = xb_ref[...]
        lane = lax.broadcasted_iota(jnp.int32, (1, E), 1)
        for j in range(EPG):
            wgu_ref[j, :, :F] = wg_ref[j].astype(jnp.bfloat16)
            wgu_ref[j, :, F:] = wu_ref[j].astype(jnp.bfloat16)
        for j in range(EPG):
            e = step * EPG + j
            gu = jnp.dot(xb, wgu_ref[j], preferred_element_type=jnp.float32)
            g = gu[:, :F]
            u = gu[:, F:]
            c = jnp.sum(combine_ref[...] * (lane == e).astype(jnp.float32),
                        axis=-1, keepdims=True)
            h_ref[parity, :, j * F:(j + 1) * F] = (
                (g / (1.0 + jnp.exp(-g))) * u * c).astype(jnp.bfloat16)

    @pl.when(step > 0)
    def _down():
        wd = wd_ref[...].reshape(EPG * F, D).astype(jnp.bfloat16)
        y = jnp.dot(h_ref[1 - parity], wd, preferred_element_type=jnp.float32)
        out_ref[...] += y


@functools.partial(jax.jit, static_argnames=())
def kernel(hidden_states, router_w, w_gate, w_up, w_down):
    x = hidden_states.reshape(-1, D)
    last = GRID - 1
    out = pl.pallas_call(
        _moe_body,
        grid=(GRID + 1,),
        in_specs=[
            pl.BlockSpec((T, D), lambda s: (0, 0)),
            pl.BlockSpec((D, E), lambda s: (0, 0)),
            pl.BlockSpec((EPG, D, F), lambda s: (jnp.minimum(s, last), 0, 0)),
            pl.BlockSpec((EPG, D, F), lambda s: (jnp.minimum(s, last), 0, 0)),
            pl.BlockSpec((EPG, F, D),
                         lambda s: (jnp.maximum(s - 1, 0), 0, 0)),
        ],
        out_specs=pl.BlockSpec((T, D), lambda s: (0, 0)),
        out_shape=jax.ShapeDtypeStruct((T, D), jnp.float32),
        scratch_shapes=[
            pltpu.VMEM((T, E), jnp.float32),
            pltpu.VMEM((T, D), jnp.bfloat16),
            pltpu.VMEM((2, T, EPG * F), jnp.bfloat16),
            pltpu.VMEM((EPG, D, 2 * F), jnp.bfloat16),
        ],
    )(x, router_w, w_gate, w_up, w_down)
    return out.reshape(hidden_states.shape)


# speedup vs baseline: 1.0297x; 1.0150x over previous
import functools

import jax
import jax.numpy as jnp
from jax import lax
from jax.experimental import pallas as pl
from jax.experimental.pallas import tpu as pltpu

E = 64
TOPK = 8
D = 768
F = 256
T = 1024
EPG = 4
GRID = E // EPG


def _moe_body(x_ref, rw_ref, wg_ref, wu_ref, wd_ref, out_ref, combine_ref,
              xb_ref, h_ref, wgu_ref):
    step = pl.program_id(0)

    @pl.when(step == 0)
    def _routing():
        x = x_ref[...]
        xb_ref[...] = x.astype(jnp.bfloat16)
        logits = jnp.dot(x, rw_ref[...], preferred_element_type=jnp.float32)
        m = jnp.max(logits, axis=-1, keepdims=True)
        ex = jnp.exp(logits - m)
        probs = ex / jnp.sum(ex, axis=-1, keepdims=True)

        p = probs
        mx = None
        for _ in range(TOPK):
            mx = jnp.max(p, axis=-1, keepdims=True)
            p = jnp.where(p >= mx, -jnp.inf, p)
        sel_w = jnp.where(probs >= mx, probs, 0.0)
        denom = jnp.sum(sel_w, axis=-1, keepdims=True)
        combine_ref[...] = sel_w / denom
        out_ref[...] = jnp.zeros((T, D), jnp.float32)

    parity = lax.rem(step, 2)

    @pl.when(step < GRID)
    def _gate_up():
        xb = xb_ref[...]
        lane = lax.broadcasted_iota(jnp.int32, (1, E), 1)
        for j in range(EPG):
            wgu_ref[j, :, :F] = wg_ref[j].astype(jnp.bfloat16)
            wgu_ref[j, :, F:] = wu_ref[j].astype(jnp.bfloat16)
        for j in range(EPG):
            e = step * EPG + j
            gu = jnp.dot(xb, wgu_ref[j], preferred_element_type=jnp.float32)
            g = gu[:, :F]
            u = gu[:, F:]
            c = jnp.sum(combine_ref[...] * (lane == e).astype(jnp.float32),
                        axis=-1, keepdims=True)
            h_ref[parity, :, j * F:(j + 1) * F] = (
                (g / (1.0 + jnp.exp(-g))) * u * c).astype(jnp.bfloat16)

    @pl.when(step > 0)
    def _down():
        wd = wd_ref[...].reshape(EPG * F, D).astype(jnp.bfloat16)
        y = jnp.dot(h_ref[1 - parity], wd, preferred_element_type=jnp.float32)
        out_ref[...] += y


@functools.partial(jax.jit, static_argnames=())
def kernel(hidden_states, router_w, w_gate, w_up, w_down):
    x = hidden_states.reshape(-1, D)
    last = GRID - 1
    out = pl.pallas_call(
        _moe_body,
        grid=(GRID + 1,),
        in_specs=[
            pl.BlockSpec((T, D), lambda s: (0, 0)),
            pl.BlockSpec((D, E), lambda s: (0, 0)),
            pl.BlockSpec((EPG, D, F), lambda s: (jnp.minimum(s, last), 0, 0)),
            pl.BlockSpec((EPG, D, F), lambda s: (jnp.minimum(s, last), 0, 0)),
            pl.BlockSpec((EPG, F, D),
                         lambda s: (jnp.maximum(s - 1, 0), 0, 0)),
        ],
        out_specs=pl.BlockSpec((T, D), lambda s: (0, 0)),
        out_shape=jax.ShapeDtypeStruct((T, D), jnp.float32),
        scratch_shapes=[
            pltpu.VMEM((T, E), jnp.float32),
            pltpu.VMEM((T, D), jnp.bfloat16),
            pltpu.VMEM((2, T, EPG * F), jnp.bfloat16),
            pltpu.VMEM((EPG, D, 2 * F), jnp.bfloat16),
        ],
    )(x, router_w, w_gate, w_up, w_down)
    return out.reshape(hidden_states.shape)
